# baseline (speedup 1.0000x reference)
import functools
import math

import jax
import jax.numpy as jnp
from jax import lax
from jax.experimental import pallas as pl
from jax.experimental.pallas import tpu as pltpu
from jax.experimental.pallas import tpu_sc as plsc

D_MODEL = 128
MAX_LEN = 2048

_NUM_CORES = 2
_NUM_SUBCORES = 16
_NUM_WORKERS = _NUM_CORES * _NUM_SUBCORES

_NBUF = 5
_LOOKAHEAD = 2


def _table_body(div_full_ref, out_ref):
    pos = lax.broadcasted_iota(jnp.int32, (MAX_LEN, D_MODEL), 0).astype(jnp.float32)
    angles = pos * div_full_ref[...]
    lane = lax.broadcasted_iota(jnp.int32, (MAX_LEN, D_MODEL), 1)
    out_ref[...] = jnp.where(lane % 2 == 0, jnp.sin(angles), jnp.cos(angles))


def _build_table(div_term):
    div_full = jnp.repeat(div_term, 2).reshape(1, D_MODEL)
    return pl.pallas_call(
        _table_body,
        out_shape=jax.ShapeDtypeStruct((MAX_LEN, D_MODEL), jnp.float32),
    )(div_full)


def _make_gather(batch, chunk):
    b_per_w = batch // _NUM_WORKERS
    n_chunks = b_per_w // chunk
    assert n_chunks % _NBUF == 0 and n_chunks >= 2 * _NBUF
    mesh = plsc.VectorSubcoreMesh(core_axis_name="c", subcore_axis_name="s")

    @functools.partial(
        pl.kernel,
        mesh=mesh,
        out_type=jax.ShapeDtypeStruct((batch, D_MODEL), jnp.float32),
        scratch_types=[
            pltpu.VMEM((b_per_w,), jnp.int32),
            pltpu.VMEM((_NBUF, chunk, D_MODEL), jnp.float32),
            pltpu.VMEM_SHARED((MAX_LEN, D_MODEL), jnp.float32),
            [pltpu.SemaphoreType.DMA] * _NBUF,
            [pltpu.SemaphoreType.DMA] * _NBUF,
        ],
    )
    def gather(table_hbm, idx_hbm, out_hbm, idx_v, rows_v, table_sp,
               gsems, osems):
        wid = lax.axis_index("s") * _NUM_CORES + lax.axis_index("c")
        base = wid * b_per_w

        @pl.when(lax.axis_index("s") == 0)
        def _():
            pltpu.sync_copy(table_hbm, table_sp)

        pltpu.sync_copy(idx_hbm.at[pl.ds(base, b_per_w)], idx_v)
        plsc.subcore_barrier()

        def gather_start(j, b):
            pltpu.async_copy(
                table_sp.at[idx_v.at[pl.ds(j * chunk, chunk)]],
                rows_v.at[b], gsems[b], priority=1)

        def gather_desc(j, b):
            return pltpu.make_async_copy(
                table_sp.at[idx_v.at[pl.ds(j * chunk, chunk)]],
                rows_v.at[b], gsems[b])

        def out_desc(j, b):
            return pltpu.make_async_copy(
                rows_v.at[b], out_hbm.at[pl.ds(base + j * chunk, chunk)],
                osems[b])

        for j in range(_LOOKAHEAD):
            gather_start(j, j % _NBUF)

        def step(j, jd, b, bd):
            @pl.when(jd >= _NBUF)
            def _():
                out_desc(jd - _NBUF, bd).wait()

            @pl.when(jd < n_chunks)
            def _():
                gather_start(jd, bd)

            gather_desc(j, b).wait()
            out_desc(j, b).start()

        def body(j2, carry):
            for u in range(_NBUF):
                j = j2 * _NBUF + u
                jd = j + _LOOKAHEAD
                step(j, jd, u, (u + _LOOKAHEAD) % _NBUF)
            return carry

        lax.fori_loop(0, n_chunks // _NBUF, body, 0)

        for j in range(n_chunks - _NBUF + _LOOKAHEAD, n_chunks):
            out_desc(j, j % _NBUF).wait()

    return gather


def kernel(position, div_term):
    table = _build_table(div_term)
    idx = position.reshape(-1)
    batch = idx.shape[0]
    gather = _make_gather(batch, chunk=128)
    return gather(table, idx)


# speedup vs baseline: 1.0004x; 1.0004x over previous
import functools
import math

import jax
import jax.numpy as jnp
from jax import lax
from jax.experimental import pallas as pl
from jax.experimental.pallas import tpu as pltpu
from jax.experimental.pallas import tpu_sc as plsc

D_MODEL = 128
MAX_LEN = 2048

_NUM_CORES = 2
_NUM_SUBCORES = 16
_NUM_WORKERS = _NUM_CORES * _NUM_SUBCORES

_NBUF = 5
_LOOKAHEAD = 2


def _table_body(div_full_ref, out_ref):
    pos = lax.broadcasted_iota(jnp.int32, (MAX_LEN, D_MODEL), 0).astype(jnp.float32)
    angles = pos * div_full_ref[...]
    lane = lax.broadcasted_iota(jnp.int32, (MAX_LEN, D_MODEL), 1)
    out_ref[...] = jnp.where(lane % 2 == 0, jnp.sin(angles), jnp.cos(angles))


def _build_table(div_term):
    div_full = jnp.repeat(div_term, 2).reshape(1, D_MODEL)
    return pl.pallas_call(
        _table_body,
        out_shape=jax.ShapeDtypeStruct((MAX_LEN, D_MODEL), jnp.float32),
    )(div_full)


def _make_gather(batch, chunk):
    b_per_w = batch // _NUM_WORKERS
    n_chunks = b_per_w // chunk
    assert n_chunks % _NBUF == 0 and n_chunks >= 2 * _NBUF
    mesh = plsc.VectorSubcoreMesh(core_axis_name="c", subcore_axis_name="s")

    @functools.partial(
        pl.kernel,
        mesh=mesh,
        out_type=jax.ShapeDtypeStruct((batch, D_MODEL), jnp.float32),
        scratch_types=[
            pltpu.VMEM((b_per_w,), jnp.int32),
            pltpu.VMEM((_NBUF, chunk, D_MODEL), jnp.float32),
            pltpu.VMEM_SHARED((MAX_LEN, D_MODEL), jnp.float32),
            [pltpu.SemaphoreType.DMA] * _NBUF,
            [pltpu.SemaphoreType.DMA] * _NBUF,
        ],
    )
    def gather(table_hbm, idx_hbm, out_hbm, idx_v, rows_v, table_sp,
               gsems, osems):
        wid = lax.axis_index("s") * _NUM_CORES + lax.axis_index("c")
        base = wid * b_per_w

        @pl.when(lax.axis_index("s") == 0)
        def _():
            pltpu.sync_copy(table_hbm, table_sp)

        pltpu.sync_copy(idx_hbm.at[pl.ds(base, b_per_w)], idx_v)
        plsc.subcore_barrier()

        def gather_desc(j, b):
            return pltpu.make_async_copy(
                table_sp.at[idx_v.at[pl.ds(j * chunk, chunk)]],
                rows_v.at[b], gsems[b])

        def out_desc(j, b):
            return pltpu.make_async_copy(
                rows_v.at[b], out_hbm.at[pl.ds(base + j * chunk, chunk)],
                osems[b])

        for j in range(_LOOKAHEAD):
            gather_desc(j, j % _NBUF).start()

        def step(j, jd, b, bd):
            @pl.when(jd >= _NBUF)
            def _():
                out_desc(jd - _NBUF, bd).wait()

            @pl.when(jd < n_chunks)
            def _():
                gather_desc(jd, bd).start()

            gather_desc(j, b).wait()
            out_desc(j, b).start()

        def body(j2, carry):
            for u in range(_NBUF):
                j = j2 * _NBUF + u
                jd = j + _LOOKAHEAD
                step(j, jd, u, (u + _LOOKAHEAD) % _NBUF)
            return carry

        lax.fori_loop(0, n_chunks // _NBUF, body, 0)

        for j in range(n_chunks - _NBUF + _LOOKAHEAD, n_chunks):
            out_desc(j, j % _NBUF).wait()

    return gather


def kernel(position, div_term):
    table = _build_table(div_term)
    idx = position.reshape(-1)
    batch = idx.shape[0]
    gather = _make_gather(batch, chunk=128)
    return gather(table, idx)
